# Initial kernel scaffold; baseline (speedup 1.0000x reference)
import jax
import jax.numpy as jnp
from jax.experimental import pallas as pl


def kernel(x, W_write, W_gate, W_out, W_beta, W_alpha, dt_bias, A_log):
    raise NotImplementedError("write your pallas kernel here")


# speedup vs baseline: 1.0221x; 1.0221x over previous
import jax
import jax.numpy as jnp
from jax.experimental import pallas as pl
from jax.experimental.pallas import tpu as pltpu

_C = 64


def _softplus(z):
    return jnp.maximum(z, 0.0) + jnp.log1p(jnp.exp(-jnp.abs(z)))


def _sigmoid(z):
    return 1.0 / (1.0 + jnp.exp(-z))


def _block_kernel(x_ref, wwT_ref, wgT_ref, woT_ref, wbT_ref, waT_ref,
                  dtb_ref, nega_ref, out_ref, S_ref, prev_ref):
    C = _C
    H, d, _ = S_ref.shape
    n = pl.program_id(1)

    @pl.when(n == 0)
    def _init():
        S_ref[...] = jnp.zeros_like(S_ref)
        prev_ref[...] = jnp.zeros_like(prev_ref)

    xb = x_ref[0]
    f32 = jnp.float32

    v_full = jnp.dot(xb, wwT_ref[...], preferred_element_type=f32)
    beta = _sigmoid(jnp.dot(xb, wbT_ref[...], preferred_element_type=f32))
    gate = _sigmoid(jnp.dot(xb, wgT_ref[...], preferred_element_type=f32))
    z = jnp.dot(xb, waT_ref[...], preferred_element_type=f32) + dtb_ref[...]
    decay = nega_ref[...] * _softplus(z)

    ri = jax.lax.broadcasted_iota(jnp.int32, (C, C), 0)
    ci = jax.lax.broadcasted_iota(jnp.int32, (C, C), 1)
    eyeC = jnp.where(ri == ci, f32(1.0), f32(0.0))
    L1 = jnp.where(ri >= ci, f32(1.0), f32(0.0))
    subD = jnp.where(ri == ci + 1, f32(1.0), f32(0.0))
    slower = ri > ci
    m0 = jax.lax.broadcasted_iota(jnp.int32, (C, d), 0) == 0

    dec = jnp.dot(L1, decay, preferred_element_type=f32)
    decT = jax.lax.dot_general(dec, eyeC, (((0,), (0,)), ((), ())),
                               preferred_element_type=f32)

    outs = []
    for h in range(H):
        xh = xb[:, h * d:(h + 1) * d]
        inv = 1.0 / jnp.maximum(jnp.sqrt(jnp.sum(xh * xh, axis=1, keepdims=True)),
                                f32(1e-12))
        rk = xh * inv
        prev_h = prev_ref[h:h + 1, :]
        wk = jnp.dot(subD, rk, preferred_element_type=f32) \
            + jnp.where(m0, jnp.broadcast_to(prev_h, (C, d)), f32(0.0))
        prev_ref[h:h + 1, :] = rk[C - 1:C, :]

        b_h = beta[:, h:h + 1]
        dec_h = dec[:, h:h + 1]
        dec_row = decT[h:h + 1, :]
        dec_exp = jnp.exp(dec_h)
        diff = dec_h - dec_row
        L = jnp.exp(jnp.where(ri >= ci, diff, f32(-1e30)))

        vh = v_full[:, h * d:(h + 1) * d] * b_h
        wkb = wk * b_h

        raw = jax.lax.dot_general(wkb, wk, (((1,), (1,)), ((), ())),
                                  preferred_element_type=f32)
        M = jnp.where(slower, raw * L, f32(0.0))
        P = -M
        A = eyeC + P
        Q = jnp.dot(P, P, preferred_element_type=f32)
        for _ in range(5):
            A = A + jnp.dot(A, Q, preferred_element_type=f32)
            Q = jnp.dot(Q, Q, preferred_element_type=f32)

        v2 = jnp.dot(A, vh, preferred_element_type=f32)
        wkcd = jnp.dot(A, wkb * dec_exp, preferred_element_type=f32)
        Sh = S_ref[h]
        v_new = v2 - jnp.dot(wkcd, Sh, preferred_element_type=f32)
        attn = jax.lax.dot_general(rk, wk, (((1,), (1,)), ((), ())),
                                   preferred_element_type=f32) * L
        o_h = jnp.dot(rk * dec_exp, Sh, preferred_element_type=f32) \
            + jnp.dot(attn, v_new, preferred_element_type=f32)

        last = dec_h[C - 1:C, :]
        dw = jnp.exp(last - dec_h)
        S_ref[h] = Sh * jnp.exp(last) + jax.lax.dot_general(
            wk * dw, v_new, (((0,), (0,)), ((), ())),
            preferred_element_type=f32)

        outs.append(o_h * gate[:, h:h + 1])

    o_full = jnp.concatenate(outs, axis=1)
    out_ref[0] = xb + jnp.dot(o_full, woT_ref[...], preferred_element_type=f32)


def kernel(x, W_write, W_gate, W_out, W_beta, W_alpha, dt_bias, A_log):
    B, T, D = x.shape
    H = A_log.shape[0]
    d = D // H
    C = _C
    N = T // C

    wwT = W_write.T
    woT = W_out.T
    wgT = W_gate.T
    wbT = W_beta.T
    waT = W_alpha.T
    dtb = dt_bias.reshape(1, H).astype(jnp.float32)
    nega = (-jnp.exp(A_log)).reshape(1, H).astype(jnp.float32)

    const = lambda b, n: (0, 0)
    out = pl.pallas_call(
        _block_kernel,
        grid=(B, N),
        in_specs=[
            pl.BlockSpec((1, C, D), lambda b, n: (b, n, 0)),
            pl.BlockSpec((D, D), const),
            pl.BlockSpec((D, H), const),
            pl.BlockSpec((D, D), const),
            pl.BlockSpec((D, H), const),
            pl.BlockSpec((D, H), const),
            pl.BlockSpec((1, H), const),
            pl.BlockSpec((1, H), const),
        ],
        out_specs=pl.BlockSpec((1, C, D), lambda b, n: (b, n, 0)),
        out_shape=jax.ShapeDtypeStruct((B, T, D), jnp.float32),
        scratch_shapes=[
            pltpu.VMEM((H, d, d), jnp.float32),
            pltpu.VMEM((H, d), jnp.float32),
        ],
        compiler_params=pltpu.CompilerParams(
            dimension_semantics=("parallel", "arbitrary"),
        ),
    )(x.astype(jnp.float32), wwT, wgT, woT, wbT, waT, dtb, nega)
    return out.astype(x.dtype)


# speedup vs baseline: 2.6678x; 2.6100x over previous
import jax
import jax.numpy as jnp
from jax.experimental import pallas as pl
from jax.experimental.pallas import tpu as pltpu

_C = 64


def _softplus(z):
    return jnp.maximum(z, 0.0) + jnp.log1p(jnp.exp(-jnp.abs(z)))


def _sigmoid(z):
    return 1.0 / (1.0 + jnp.exp(-z))


def _block_kernel(xp_ref, xn_ref, wwT_ref, wcat_ref, woT_ref, dtb_ref, nega_ref,
                  out_ref, S_ref, prev_ref, v2_ref, wkcd_ref, rkdec_ref,
                  wkdw_ref, attn_ref, gate_ref, sdec_ref):
    C = _C
    H, d, _ = S_ref.shape
    D = H * d
    f32 = jnp.float32
    k = pl.program_id(1)

    @pl.when(k == 0)
    def _init_prev():
        prev_ref[...] = jnp.zeros_like(prev_ref)

    @pl.when(k <= 1)
    def _init_S():
        S_ref[...] = jnp.zeros_like(S_ref)

    xb_prev = xp_ref[0]
    outs = []
    for h in range(H):
        sl = slice(h * d, (h + 1) * d)
        v2 = v2_ref[:, sl]
        wkcd = wkcd_ref[:, sl]
        rkdec = rkdec_ref[:, sl]
        wkdw = wkdw_ref[:, sl]
        attn = attn_ref[h * C:(h + 1) * C, :]
        Sh = S_ref[h]
        both = jnp.dot(jnp.concatenate([wkcd, rkdec], axis=0), Sh,
                       preferred_element_type=f32)
        v_new = v2 - both[:C]
        o_h = both[C:] + jnp.dot(attn, v_new, preferred_element_type=f32)
        S_ref[h] = Sh * sdec_ref[:, h:h + 1] + jax.lax.dot_general(
            wkdw, v_new, (((0,), (0,)), ((), ())), preferred_element_type=f32)
        outs.append(o_h * gate_ref[:, h:h + 1])
    o_full = jnp.concatenate(outs, axis=1)
    out_ref[0] = xb_prev + jnp.dot(o_full, woT_ref[...],
                                   preferred_element_type=f32)

    xb = xn_ref[0]
    proj = jnp.dot(xb, wcat_ref[...], preferred_element_type=f32)
    beta = _sigmoid(proj[:, 0:H])
    gate = _sigmoid(proj[:, H:2 * H])
    z = proj[:, 2 * H:3 * H] + dtb_ref[...]
    decay = nega_ref[...] * _softplus(z)
    v_full = jnp.dot(xb, wwT_ref[...], preferred_element_type=f32)

    ri = jax.lax.broadcasted_iota(jnp.int32, (C, C), 0)
    ci = jax.lax.broadcasted_iota(jnp.int32, (C, C), 1)
    eyeC = jnp.where(ri == ci, f32(1.0), f32(0.0))
    L1 = jnp.where(ri >= ci, f32(1.0), f32(0.0))
    subD = jnp.where(ri == ci + 1, f32(1.0), f32(0.0))
    m0 = jax.lax.broadcasted_iota(jnp.int32, (C, D), 0) == 0
    riS = jax.lax.broadcasted_iota(jnp.int32, (C, H * C), 0)
    ciS = jax.lax.broadcasted_iota(jnp.int32, (C, H * C), 1)
    eyeS = jnp.where((ciS & (C - 1)) == riS, f32(1.0), f32(0.0))
    rB = jax.lax.broadcasted_iota(jnp.int32, (H * C, H * C), 0)
    cB = jax.lax.broadcasted_iota(jnp.int32, (H * C, H * C), 1)
    blk = (rB >> 6) == (cB >> 6)

    dec = jnp.dot(L1, decay, preferred_element_type=f32)
    decT = jax.lax.dot_general(dec, eyeC, (((0,), (0,)), ((), ())),
                               preferred_element_type=f32)

    rks = []
    for h in range(H):
        xh = xb[:, h * d:(h + 1) * d]
        inv = 1.0 / jnp.maximum(
            jnp.sqrt(jnp.sum(xh * xh, axis=1, keepdims=True)), f32(1e-12))
        rks.append(xh * inv)
    rk_all = jnp.concatenate(rks, axis=1)
    prev_row = prev_ref[...]
    wk_all = jnp.dot(subD, rk_all, preferred_element_type=f32) \
        + jnp.where(m0, jnp.broadcast_to(prev_row, (C, D)), f32(0.0))
    prev_ref[...] = rk_all[C - 1:C, :]

    Ms, Ls, wkbs = [], [], []
    for h in range(H):
        sl = slice(h * d, (h + 1) * d)
        rk = rk_all[:, sl]
        wk = wk_all[:, sl]
        wkb = wk * beta[:, h:h + 1]
        rawb = jax.lax.dot_general(
            jnp.concatenate([wkb, rk], axis=0), wk,
            (((1,), (1,)), ((), ())), preferred_element_type=f32)
        L = jnp.exp(jnp.where(ri >= ci, dec[:, h:h + 1] - decT[h:h + 1, :],
                              f32(-1e30)))
        Ms.append(jnp.where(ri > ci, rawb[:C] * L, f32(0.0)))
        attn_ref[h * C:(h + 1) * C, :] = rawb[C:] * L
        Ls.append(L)
        wkbs.append(wkb)

    P = -jnp.concatenate(Ms, axis=1)

    def bd(Q):
        return jnp.where(blk, pltpu.repeat(Q, H, axis=0), f32(0.0))

    A = eyeS + P
    Q = jnp.dot(P, bd(P), preferred_element_type=f32)
    for i in range(5):
        Qb = bd(Q)
        if i < 4:
            both2 = jnp.dot(jnp.concatenate([A, Q], axis=0), Qb,
                            preferred_element_type=f32)
            A = A + both2[:C]
            Q = both2[C:]
        else:
            A = A + jnp.dot(A, Qb, preferred_element_type=f32)

    sdec_ref[...] = jnp.exp(dec[C - 1:C, :])
    gate_ref[...] = gate
    for h in range(H):
        sl = slice(h * d, (h + 1) * d)
        A_h = A[:, h * C:(h + 1) * C]
        dec_h = dec[:, h:h + 1]
        dec_exp = jnp.exp(dec_h)
        vh = v_full[:, sl] * beta[:, h:h + 1]
        rhs = jnp.concatenate([vh, wkbs[h] * dec_exp], axis=1)
        res = jnp.dot(A_h, rhs, preferred_element_type=f32)
        v2_ref[:, sl] = res[:, :d]
        wkcd_ref[:, sl] = res[:, d:]
        rkdec_ref[:, sl] = rk_all[:, sl] * dec_exp
        dw = jnp.exp(dec_h[C - 1:C, :] - dec_h)
        wkdw_ref[:, sl] = wk_all[:, sl] * dw


def kernel(x, W_write, W_gate, W_out, W_beta, W_alpha, dt_bias, A_log):
    B, T, D = x.shape
    H = A_log.shape[0]
    d = D // H
    C = _C
    N = T // C

    wwT = W_write.T
    woT = W_out.T
    wcat = jnp.concatenate([W_beta.T, W_gate.T, W_alpha.T], axis=1)
    dtb = dt_bias.reshape(1, H).astype(jnp.float32)
    nega = (-jnp.exp(A_log)).reshape(1, H).astype(jnp.float32)

    const = lambda b, k: (0, 0)
    out = pl.pallas_call(
        _block_kernel,
        grid=(B, N + 1),
        in_specs=[
            pl.BlockSpec((1, C, D), lambda b, k: (b, jnp.maximum(k - 1, 0), 0)),
            pl.BlockSpec((1, C, D), lambda b, k: (b, jnp.minimum(k, N - 1), 0)),
            pl.BlockSpec((D, D), const),
            pl.BlockSpec((D, 3 * H), const),
            pl.BlockSpec((D, D), const),
            pl.BlockSpec((1, H), const),
            pl.BlockSpec((1, H), const),
        ],
        out_specs=pl.BlockSpec((1, C, D), lambda b, k: (b, jnp.maximum(k - 1, 0), 0)),
        out_shape=jax.ShapeDtypeStruct((B, T, D), jnp.float32),
        scratch_shapes=[
            pltpu.VMEM((H, d, d), jnp.float32),
            pltpu.VMEM((1, D), jnp.float32),
            pltpu.VMEM((C, D), jnp.float32),
            pltpu.VMEM((C, D), jnp.float32),
            pltpu.VMEM((C, D), jnp.float32),
            pltpu.VMEM((C, D), jnp.float32),
            pltpu.VMEM((H * C, C), jnp.float32),
            pltpu.VMEM((C, H), jnp.float32),
            pltpu.VMEM((1, H), jnp.float32),
        ],
        compiler_params=pltpu.CompilerParams(
            dimension_semantics=("parallel", "arbitrary"),
        ),
    )(x.astype(jnp.float32), x.astype(jnp.float32), wwT, wcat, woT, dtb, nega)
    return out.astype(x.dtype)
